# baseline (speedup 1.0000x reference)
import functools

import jax
import jax.numpy as jnp
from jax import lax
from jax.experimental import pallas as pl
from jax.experimental.pallas import tpu as pltpu
from jax.experimental.pallas import tpu_sc as plsc

N = 10000
E = 320000
D = 128
NP = 10112

NC = 2
NS = 16
NW = NC * NS
CHUNK = 128
NCHUNKS = E // CHUNK
BASE_CH = NCHUNKS // NW
EXTRA_CH = NCHUNKS - BASE_CH * NW
RPT = NP // NS
ZROWS = 104


def _sc_agg_body(x_hbm, ei_hbm, agg_out,
                 src_v, dst_v, rows_v, zagg_v, agg_sh, gsem0, gsem1):
    c = lax.axis_index("c")
    s = lax.axis_index("s")
    wid = c * NS + s
    zeros16 = jnp.zeros((16,), jnp.float32)
    gsems = (gsem0, gsem1)

    def zfill_agg(i, carry):
        zagg_v[i // 8, pl.ds((i % 8) * 16, 16)] = zeros16
        return carry
    lax.fori_loop(0, ZROWS * 8, zfill_agg, 0)

    row0 = pl.multiple_of(s * RPT, 8)
    for k in range(6):
        pltpu.sync_copy(zagg_v, agg_sh.at[pl.ds(row0 + k * ZROWS, ZROWS)])
    pltpu.sync_copy(zagg_v.at[pl.ds(0, 8)], agg_sh.at[pl.ds(row0 + 6 * ZROWS, 8)])
    plsc.subcore_barrier()

    def e_of(j):
        return pl.multiple_of((wid + j * NW) * CHUNK, CHUNK)

    def load_and_gather(b, j):
        e0 = e_of(j)
        pltpu.sync_copy(ei_hbm.at[0, pl.ds(e0, CHUNK)], src_v.at[b])
        pltpu.sync_copy(ei_hbm.at[1, pl.ds(e0, CHUNK)], dst_v.at[b])
        pltpu.async_copy(x_hbm.at[src_v.at[b]], rows_v.at[b], gsems[b])

    def wait_gather(b):
        pltpu.make_async_copy(x_hbm.at[src_v.at[b]], rows_v.at[b],
                              gsems[b]).wait()

    def scatter(b):
        pltpu.sync_copy(rows_v.at[b], agg_sh.at[dst_v.at[b]], add=True)

    nch = BASE_CH + 1

    load_and_gather(0, 0)
    load_and_gather(1, 1)

    def body(j2, carry):
        for b in range(2):
            j = j2 * 2 + b
            wait_gather(b)
            scatter(b)

            @pl.when(j + 2 < nch - 1 + jnp.where(wid < EXTRA_CH, 1, 0))
            def _next():
                load_and_gather(b, j + 2)
        return carry
    lax.fori_loop(0, BASE_CH // 2, body, 0)

    @pl.when(wid < EXTRA_CH)
    def _extra():
        wait_gather(0)
        scatter(0)

    plsc.subcore_barrier()
    pltpu.sync_copy(agg_sh.at[pl.ds(row0, RPT)], agg_out.at[c, pl.ds(row0, RPT)])


_sc_agg = functools.partial(
    pl.kernel,
    mesh=plsc.VectorSubcoreMesh(core_axis_name="c", subcore_axis_name="s"),
    out_type=[jax.ShapeDtypeStruct((NC, NP, D), jnp.float32)],
    scratch_types=[
        pltpu.VMEM((2, CHUNK), jnp.int32),
        pltpu.VMEM((2, CHUNK), jnp.int32),
        pltpu.VMEM((2, CHUNK, D), jnp.float32),
        pltpu.VMEM((ZROWS, D), jnp.float32),
        pltpu.VMEM_SHARED((NP, D), jnp.float32),
        pltpu.SemaphoreType.DMA,
        pltpu.SemaphoreType.DMA,
    ],
)(_sc_agg_body)


def _sc_cnt_body(ei_hbm, cnt_out,
                 dstb_v, dsti_v, ones_v, cnt_sh):
    c = lax.axis_index("c")
    s = lax.axis_index("s")
    wid = c * NS + s
    zeros16 = jnp.zeros((16,), jnp.float32)
    ones16 = jnp.ones((16,), jnp.float32)

    def zfill(i, carry):
        ones_v[i // 8, pl.ds((i % 8) * 16, 16)] = zeros16
        return carry
    lax.fori_loop(0, ZROWS * 8, zfill, 0)

    row0 = pl.multiple_of(s * RPT, 8)
    for k in range(6):
        pltpu.sync_copy(ones_v.at[pl.ds(0, ZROWS)],
                        cnt_sh.at[pl.ds(row0 + k * ZROWS, ZROWS)])
    pltpu.sync_copy(ones_v.at[pl.ds(0, 8)],
                    cnt_sh.at[pl.ds(row0 + 6 * ZROWS, 8)])

    def ofill(i, carry):
        ones_v[i // 8, pl.ds((i % 8) * 16, 16)] = ones16
        return carry
    lax.fori_loop(0, CHUNK * 8, ofill, 0)

    e0 = pl.multiple_of(wid * BASE_CH * CHUNK, 8)
    pltpu.sync_copy(ei_hbm.at[1, pl.ds(e0, BASE_CH * CHUNK)], dstb_v)
    plsc.subcore_barrier()

    def body(j, carry):
        for k in range(CHUNK // 16):
            dsti_v[pl.ds(k * 16, 16)] = dstb_v[pl.ds(j * CHUNK + k * 16, 16)]
        pltpu.sync_copy(ones_v, cnt_sh.at[dsti_v], add=True)
        return carry
    lax.fori_loop(0, BASE_CH, body, 0)

    @pl.when(wid < EXTRA_CH)
    def _extra():
        et = pl.multiple_of((NW * BASE_CH + wid) * CHUNK, CHUNK)
        pltpu.sync_copy(ei_hbm.at[1, pl.ds(et, CHUNK)], dsti_v)
        pltpu.sync_copy(ones_v, cnt_sh.at[dsti_v], add=True)

    plsc.subcore_barrier()
    pltpu.sync_copy(cnt_sh.at[pl.ds(row0, RPT)], cnt_out.at[c, pl.ds(row0, RPT)])


_sc_cnt = functools.partial(
    pl.kernel,
    mesh=plsc.VectorSubcoreMesh(core_axis_name="c", subcore_axis_name="s"),
    out_type=[jax.ShapeDtypeStruct((NC, NP, D), jnp.float32)],
    scratch_types=[
        pltpu.VMEM((BASE_CH * CHUNK,), jnp.int32),
        pltpu.VMEM((CHUNK,), jnp.int32),
        pltpu.VMEM((CHUNK, D), jnp.float32),
        pltpu.VMEM_SHARED((NP, D), jnp.float32),
    ],
)(_sc_cnt_body)


def _tc_root_body(x_ref, wr_ref, b_ref, o_ref):
    o_ref[...] = (
        jnp.dot(x_ref[...], wr_ref[...], preferred_element_type=jnp.float32)
        + b_ref[...]
    )


def _tc_rel_body(root_ref, wl_ref, agg_ref, cnt_ref, o_ref):
    agg = agg_ref[0] + agg_ref[1]
    mean = agg / jnp.maximum(cnt_ref[...], 1.0)
    o_ref[...] = root_ref[...] + jnp.dot(
        mean, wl_ref[...], preferred_element_type=jnp.float32)


_ROWS = 1000
_GRID = N // _ROWS

_tc_root = pl.pallas_call(
    _tc_root_body,
    grid=(_GRID,),
    in_specs=[
        pl.BlockSpec((_ROWS, D), lambda i: (i, 0)),
        pl.BlockSpec((D, D), lambda i: (0, 0)),
        pl.BlockSpec((1, D), lambda i: (0, 0)),
    ],
    out_specs=pl.BlockSpec((_ROWS, D), lambda i: (i, 0)),
    out_shape=jax.ShapeDtypeStruct((N, D), jnp.float32),
)

_tc_rel = pl.pallas_call(
    _tc_rel_body,
    grid=(_GRID,),
    in_specs=[
        pl.BlockSpec((_ROWS, D), lambda i: (i, 0)),
        pl.BlockSpec((D, D), lambda i: (0, 0)),
        pl.BlockSpec((NC, _ROWS, D), lambda i: (0, i, 0)),
        pl.BlockSpec((_ROWS, 1), lambda i: (i, 0)),
    ],
    out_specs=pl.BlockSpec((_ROWS, D), lambda i: (i, 0)),
    out_shape=jax.ShapeDtypeStruct((N, D), jnp.float32),
)


def kernel(x, edge_index, W_root, b_root, W_rel):
    ei = edge_index.astype(jnp.int32)
    root = _tc_root(x, W_root.T, b_root.reshape(1, D))
    (agg_part,) = _sc_agg(x, ei)
    (cnt_part,) = _sc_cnt(ei)
    cnt_col = (cnt_part[0, :N, 0] + cnt_part[1, :N, 0])[:, None]
    return _tc_rel(root, W_rel.T, agg_part[:, :N], cnt_col)


# speedup vs baseline: 1.0083x; 1.0083x over previous
import functools

import jax
import jax.numpy as jnp
from jax import lax
from jax.experimental import pallas as pl
from jax.experimental.pallas import tpu as pltpu
from jax.experimental.pallas import tpu_sc as plsc

N = 10000
E = 320000
D = 128
NP = 10112

NC = 2
NS = 16
NW = NC * NS
CHUNK = 128
NCHUNKS = E // CHUNK
BASE_CH = NCHUNKS // NW
EXTRA_CH = NCHUNKS - BASE_CH * NW
RPT = NP // NS
ZROWS = 104


def _sc_agg_body(x_hbm, ei_hbm, agg_out,
                 src_v, dst_v, rows_v, zagg_v, agg_sh, gsem0, gsem1):
    c = lax.axis_index("c")
    s = lax.axis_index("s")
    wid = c * NS + s
    zeros16 = jnp.zeros((16,), jnp.float32)
    gsems = (gsem0, gsem1)

    def zfill_agg(i, carry):
        zagg_v[i // 8, pl.ds((i % 8) * 16, 16)] = zeros16
        return carry
    lax.fori_loop(0, ZROWS * 8, zfill_agg, 0)

    row0 = pl.multiple_of(s * RPT, 8)
    for k in range(6):
        pltpu.sync_copy(zagg_v, agg_sh.at[pl.ds(row0 + k * ZROWS, ZROWS)])
    pltpu.sync_copy(zagg_v.at[pl.ds(0, 8)], agg_sh.at[pl.ds(row0 + 6 * ZROWS, 8)])
    plsc.subcore_barrier()

    def e_of(j):
        return pl.multiple_of((wid + j * NW) * CHUNK, CHUNK)

    def load_and_gather(b, j):
        e0 = e_of(j)
        pltpu.sync_copy(ei_hbm.at[0, pl.ds(e0, CHUNK)], src_v.at[b])
        pltpu.sync_copy(ei_hbm.at[1, pl.ds(e0, CHUNK)], dst_v.at[b])
        pltpu.async_copy(x_hbm.at[src_v.at[b]], rows_v.at[b], gsems[b])

    def wait_gather(b):
        pltpu.make_async_copy(x_hbm.at[src_v.at[b]], rows_v.at[b],
                              gsems[b]).wait()

    def scatter(b):
        pltpu.sync_copy(rows_v.at[b], agg_sh.at[dst_v.at[b]], add=True)

    nch = BASE_CH + 1

    load_and_gather(0, 0)
    load_and_gather(1, 1)

    def body(j2, carry):
        for b in range(2):
            j = j2 * 2 + b
            wait_gather(b)
            scatter(b)

            @pl.when(j + 2 < nch - 1 + jnp.where(wid < EXTRA_CH, 1, 0))
            def _next():
                load_and_gather(b, j + 2)
        return carry
    lax.fori_loop(0, BASE_CH // 2, body, 0)

    @pl.when(wid < EXTRA_CH)
    def _extra():
        wait_gather(0)
        scatter(0)

    plsc.subcore_barrier()
    pltpu.sync_copy(agg_sh.at[pl.ds(row0, RPT)], agg_out.at[c, pl.ds(row0, RPT)])


_sc_agg = functools.partial(
    pl.kernel,
    mesh=plsc.VectorSubcoreMesh(core_axis_name="c", subcore_axis_name="s"),
    out_type=[jax.ShapeDtypeStruct((NC, NP, D), jnp.float32)],
    scratch_types=[
        pltpu.VMEM((2, CHUNK), jnp.int32),
        pltpu.VMEM((2, CHUNK), jnp.int32),
        pltpu.VMEM((2, CHUNK, D), jnp.float32),
        pltpu.VMEM((ZROWS, D), jnp.float32),
        pltpu.VMEM_SHARED((NP, D), jnp.float32),
        pltpu.SemaphoreType.DMA,
        pltpu.SemaphoreType.DMA,
    ],
)(_sc_agg_body)


def _sc_cnt_body(ei_hbm, cnt_out,
                 dst_v, ones_v, cnt_sh, isem0, isem1):
    c = lax.axis_index("c")
    s = lax.axis_index("s")
    wid = c * NS + s
    zeros16 = jnp.zeros((16,), jnp.float32)
    ones16 = jnp.ones((16,), jnp.float32)

    def zfill(i, carry):
        ones_v[i // 8, pl.ds((i % 8) * 16, 16)] = zeros16
        return carry
    lax.fori_loop(0, ZROWS * 8, zfill, 0)

    row0 = pl.multiple_of(s * RPT, 8)
    for k in range(6):
        pltpu.sync_copy(ones_v.at[pl.ds(0, ZROWS)],
                        cnt_sh.at[pl.ds(row0 + k * ZROWS, ZROWS)])
    pltpu.sync_copy(ones_v.at[pl.ds(0, 8)],
                    cnt_sh.at[pl.ds(row0 + 6 * ZROWS, 8)])

    def ofill(i, carry):
        ones_v[i // 8, pl.ds((i % 8) * 16, 16)] = ones16
        return carry
    lax.fori_loop(0, CHUNK * 8, ofill, 0)
    plsc.subcore_barrier()

    isems = (isem0, isem1)

    def e_of(j):
        return pl.multiple_of((wid + j * NW) * CHUNK, CHUNK)

    def load_idx(b, j):
        pltpu.async_copy(ei_hbm.at[1, pl.ds(e_of(j), CHUNK)], dst_v.at[b],
                         isems[b])

    def wait_idx(b, j):
        pltpu.make_async_copy(ei_hbm.at[1, pl.ds(e_of(j), CHUNK)],
                              dst_v.at[b], isems[b]).wait()

    load_idx(0, 0)
    load_idx(1, 1)

    def body(j2, carry):
        for b in range(2):
            j = j2 * 2 + b
            wait_idx(b, j)
            pltpu.sync_copy(ones_v, cnt_sh.at[dst_v.at[b]], add=True)

            @pl.when(j + 2 < BASE_CH + jnp.where(wid < EXTRA_CH, 1, 0))
            def _next():
                load_idx(b, j + 2)
        return carry
    lax.fori_loop(0, BASE_CH // 2, body, 0)

    @pl.when(wid < EXTRA_CH)
    def _extra():
        wait_idx(0, BASE_CH)
        pltpu.sync_copy(ones_v, cnt_sh.at[dst_v.at[0]], add=True)

    plsc.subcore_barrier()
    pltpu.sync_copy(cnt_sh.at[pl.ds(row0, RPT)], cnt_out.at[c, pl.ds(row0, RPT)])


_sc_cnt = functools.partial(
    pl.kernel,
    mesh=plsc.VectorSubcoreMesh(core_axis_name="c", subcore_axis_name="s"),
    out_type=[jax.ShapeDtypeStruct((NC, NP, D), jnp.float32)],
    scratch_types=[
        pltpu.VMEM((2, CHUNK), jnp.int32),
        pltpu.VMEM((CHUNK, D), jnp.float32),
        pltpu.VMEM_SHARED((NP, D), jnp.float32),
        pltpu.SemaphoreType.DMA,
        pltpu.SemaphoreType.DMA,
    ],
)(_sc_cnt_body)


def _tc_root_body(x_ref, wr_ref, b_ref, o_ref):
    o_ref[...] = (
        jnp.dot(x_ref[...], wr_ref[...], preferred_element_type=jnp.float32)
        + b_ref[...]
    )


def _tc_rel_body(root_ref, wl_ref, agg_ref, cnt_ref, o_ref):
    agg = agg_ref[0] + agg_ref[1]
    mean = agg / jnp.maximum(cnt_ref[...], 1.0)
    o_ref[...] = root_ref[...] + jnp.dot(
        mean, wl_ref[...], preferred_element_type=jnp.float32)


_ROWS = 1000
_GRID = N // _ROWS

_tc_root = pl.pallas_call(
    _tc_root_body,
    grid=(_GRID,),
    in_specs=[
        pl.BlockSpec((_ROWS, D), lambda i: (i, 0)),
        pl.BlockSpec((D, D), lambda i: (0, 0)),
        pl.BlockSpec((1, D), lambda i: (0, 0)),
    ],
    out_specs=pl.BlockSpec((_ROWS, D), lambda i: (i, 0)),
    out_shape=jax.ShapeDtypeStruct((N, D), jnp.float32),
)

_tc_rel = pl.pallas_call(
    _tc_rel_body,
    grid=(_GRID,),
    in_specs=[
        pl.BlockSpec((_ROWS, D), lambda i: (i, 0)),
        pl.BlockSpec((D, D), lambda i: (0, 0)),
        pl.BlockSpec((NC, _ROWS, D), lambda i: (0, i, 0)),
        pl.BlockSpec((_ROWS, 1), lambda i: (i, 0)),
    ],
    out_specs=pl.BlockSpec((_ROWS, D), lambda i: (i, 0)),
    out_shape=jax.ShapeDtypeStruct((N, D), jnp.float32),
)


def kernel(x, edge_index, W_root, b_root, W_rel):
    ei = edge_index.astype(jnp.int32)
    root = _tc_root(x, W_root.T, b_root.reshape(1, D))
    (agg_part,) = _sc_agg(x, ei)
    (cnt_part,) = _sc_cnt(ei)
    cnt_col = (cnt_part[0, :N, 0] + cnt_part[1, :N, 0])[:, None]
    return _tc_rel(root, W_rel.T, agg_part[:, :N], cnt_col)
